# Initial kernel scaffold; baseline (speedup 1.0000x reference)
import jax
import jax.numpy as jnp
from jax.experimental import pallas as pl


def kernel(L_vec):
    raise NotImplementedError("write your pallas kernel here")


# speedup vs baseline: 1.8047x; 1.8047x over previous
import functools

import jax
import jax.numpy as jnp
from jax import lax
from jax.experimental import pallas as pl
from jax.experimental.pallas import tpu as pltpu
from jax.experimental.pallas import tpu_sc as plsc

BATCH = 1024
SIZE = 128
NVEC = SIZE * (SIZE + 1) // 2
LANES = 16
NCHUNK = SIZE // LANES
NW = 32
BPW = BATCH // NW
MIN_DIAG = 1e-8

_MESH = plsc.VectorSubcoreMesh(core_axis_name="c", subcore_axis_name="s")


@functools.partial(
    pl.kernel,
    out_type=jax.ShapeDtypeStruct((BATCH, SIZE * SIZE), jnp.float32),
    mesh=_MESH,
    scratch_types=[
        pltpu.VMEM((NVEC,), jnp.float32),
        pltpu.VMEM((SIZE * SIZE,), jnp.float32),
    ],
)
def _tril_assemble(vec_hbm, out_hbm, vec_v, out_v):
    wid = lax.axis_index("s") * 2 + lax.axis_index("c")
    lane = lax.iota(jnp.int32, LANES)
    zeros = jnp.zeros((LANES,), jnp.float32)

    def zero_body(t, _):
        out_v[pl.ds(t * LANES, LANES)] = zeros
        return 0

    lax.fori_loop(0, SIZE * SIZE // LANES, zero_body, 0)

    def per_batch(k, _):
        b = wid * BPW + k
        pltpu.sync_copy(vec_hbm.at[b], vec_v)
        for c in range(NCHUNK):
            def row_body(i, tri, c=c):
                v = vec_v[pl.ds(tri + LANES * c, LANES)]
                li = i - LANES * c
                res = jnp.where(
                    lane < li, v,
                    jnp.where(lane == li, jnp.abs(v) + MIN_DIAG, 0.0),
                )
                out_v[pl.ds(SIZE * i + LANES * c, LANES)] = res
                return tri + i + 1

            lax.fori_loop(LANES * c, SIZE, row_body,
                          (LANES * c) * (LANES * c + 1) // 2)
        pltpu.sync_copy(out_v, out_hbm.at[b])
        return 0

    lax.fori_loop(0, BPW, per_batch, 0)


def kernel(L_vec):
    out = _tril_assemble(L_vec)
    return out.reshape(BATCH, SIZE, SIZE)


# speedup vs baseline: 1.9934x; 1.1045x over previous
import functools

import jax
import jax.numpy as jnp
import numpy as np
from jax import lax
from jax.experimental import pallas as pl
from jax.experimental.pallas import tpu as pltpu
from jax.experimental.pallas import tpu_sc as plsc

BATCH = 1024
SIZE = 128
NVEC = SIZE * (SIZE + 1) // 2
LANES = 16
NCHUNKS = NVEC // LANES
NW = 32
BPW = BATCH // NW
MIN_DIAG = 1e-8

_TI, _TJ = np.tril_indices(SIZE)
_DST_NP = (_TI * SIZE + _TJ).astype(np.int32)

_MESH = plsc.VectorSubcoreMesh(core_axis_name="c", subcore_axis_name="s")


@functools.partial(
    pl.kernel,
    out_type=jax.ShapeDtypeStruct((BATCH, SIZE * SIZE), jnp.float32),
    mesh=_MESH,
    scratch_types=[
        pltpu.VMEM((NVEC,), jnp.int32),
        pltpu.VMEM((NVEC,), jnp.float32),
        pltpu.VMEM((SIZE * SIZE + LANES,), jnp.float32),
    ],
    compiler_params=pltpu.CompilerParams(needs_layout_passes=False),
)
def _tril_assemble(vec_hbm, dst_hbm, out_hbm, idx_v, vec_v, out_v):
    wid = lax.axis_index("s") * 2 + lax.axis_index("c")
    lane = lax.iota(jnp.int32, LANES)
    zeros = jnp.zeros((LANES,), jnp.float32)

    pltpu.sync_copy(dst_hbm, idx_v)

    def zero_body(t, _):
        out_v[pl.ds(t * LANES, LANES)] = zeros
        return 0

    lax.fori_loop(0, SIZE * SIZE // LANES, zero_body, 0, unroll=8)

    def per_batch(k, _):
        b = wid * BPW + k
        pltpu.sync_copy(vec_hbm.at[b], vec_v)

        def chunk_body(t, _):
            off = t * LANES
            v = vec_v[pl.ds(off, LANES)]
            ix = idx_v[pl.ds(off, LANES)]
            plsc.store_scatter(out_v, [ix], v)
            return 0

        lax.fori_loop(0, NCHUNKS, chunk_body, 0, unroll=4)

        for c in range(SIZE // LANES):
            ivec = lane + LANES * c
            src = (ivec * (ivec + 3)) >> 1
            d = plsc.load_gather(vec_v, [src])
            d = jnp.abs(d) + MIN_DIAG
            plsc.store_scatter(out_v, [ivec * (SIZE + 1)], d)

        pltpu.sync_copy(out_v.at[pl.ds(0, SIZE * SIZE)], out_hbm.at[b])
        return 0

    lax.fori_loop(0, BPW, per_batch, 0)


def kernel(L_vec):
    out = _tril_assemble(L_vec, jnp.asarray(_DST_NP))
    return out.reshape(BATCH, SIZE, SIZE)


# speedup vs baseline: 2.0258x; 1.0162x over previous
import functools

import jax
import jax.numpy as jnp
import numpy as np
from jax import lax
from jax.experimental import pallas as pl
from jax.experimental.pallas import tpu as pltpu
from jax.experimental.pallas import tpu_sc as plsc

BATCH = 1024
SIZE = 128
NVEC = SIZE * (SIZE + 1) // 2
LANES = 16
NCHUNKS = NVEC // LANES
NW = 32
BPW = BATCH // NW
MIN_DIAG = 1e-8

_TI, _TJ = np.tril_indices(SIZE)
_IJ_NP = ((_TI << 8) | _TJ).astype(np.int32)

_MESH = plsc.VectorSubcoreMesh(core_axis_name="c", subcore_axis_name="s")


@functools.partial(
    pl.kernel,
    out_type=jax.ShapeDtypeStruct((BATCH, SIZE, SIZE), jnp.float32),
    mesh=_MESH,
    scratch_types=[
        pltpu.VMEM((NVEC,), jnp.int32),
        pltpu.VMEM((NVEC,), jnp.float32),
        pltpu.VMEM((SIZE, SIZE), jnp.float32),
    ],
    compiler_params=pltpu.CompilerParams(needs_layout_passes=False),
)
def _tril_assemble(vec_hbm, ij_hbm, out_hbm, idx_v, vec_v, out_v):
    wid = lax.axis_index("s") * 2 + lax.axis_index("c")
    lane = lax.iota(jnp.int32, LANES)
    zeros = jnp.zeros((LANES,), jnp.float32)

    pltpu.sync_copy(ij_hbm, idx_v)

    def zero_body(r, _):
        for c in range(SIZE // LANES):
            out_v[r, pl.ds(c * LANES, LANES)] = zeros
        return 0

    lax.fori_loop(0, SIZE, zero_body, 0)

    def per_batch(k, _):
        b = wid * BPW + k
        pltpu.sync_copy(vec_hbm.at[b], vec_v)

        def chunk_body(t, _):
            off = t * LANES
            v = vec_v[pl.ds(off, LANES)]
            ix = idx_v[pl.ds(off, LANES)]
            plsc.store_scatter(out_v, [ix >> 8, ix & 0xFF], v)
            return 0

        lax.fori_loop(0, NCHUNKS, chunk_body, 0, unroll=4)

        for c in range(SIZE // LANES):
            ivec = lane + LANES * c
            src = (ivec * (ivec + 3)) >> 1
            d = plsc.load_gather(vec_v, [src])
            d = jnp.abs(d) + MIN_DIAG
            plsc.store_scatter(out_v, [ivec, ivec], d)

        pltpu.sync_copy(out_v, out_hbm.at[b])
        return 0

    lax.fori_loop(0, BPW, per_batch, 0)


def kernel(L_vec):
    return _tril_assemble(L_vec, jnp.asarray(_IJ_NP))


# speedup vs baseline: 3.8124x; 1.8820x over previous
import functools

import jax
import jax.numpy as jnp
import numpy as np
from jax import lax
from jax.experimental import pallas as pl
from jax.experimental.pallas import tpu as pltpu
from jax.experimental.pallas import tpu_sc as plsc

BATCH = 1024
SIZE = 128
NVEC = SIZE * (SIZE + 1) // 2
LANES = 16
NCHUNKS = NVEC // LANES
NW = 32
BPW = BATCH // NW
MIN_DIAG = 1e-8

_TI, _TJ = np.tril_indices(SIZE)
_IJ_NP = ((_TI << 8) | _TJ).astype(np.int32)

_MESH = plsc.VectorSubcoreMesh(core_axis_name="c", subcore_axis_name="s")


@functools.partial(
    pl.kernel,
    out_type=jax.ShapeDtypeStruct((BATCH, SIZE, SIZE), jnp.float32),
    mesh=_MESH,
    scratch_types=[
        pltpu.VMEM((NVEC,), jnp.int32),
        pltpu.VMEM((NVEC,), jnp.float32),
        pltpu.VMEM((SIZE, SIZE), jnp.float32),
    ],
    compiler_params=pltpu.CompilerParams(needs_layout_passes=False),
)
def _tril_assemble(vec_hbm, idx_hbm, out_hbm, idx_v, vec_v, out_v):
    wid = lax.axis_index("s") * 2 + lax.axis_index("c")
    lane = lax.iota(jnp.int32, LANES)
    zeros = jnp.zeros((LANES,), jnp.float32)

    pltpu.sync_copy(idx_hbm, idx_v)

    @plsc.parallel_loop(0, SIZE, unroll=2)
    def _(r):
        for c in range(SIZE // LANES):
            out_v[r, pl.ds(c * LANES, LANES)] = zeros

    def per_batch(k, _):
        b = wid * BPW + k
        pltpu.sync_copy(vec_hbm.at[b], vec_v)

        @plsc.parallel_loop(0, NCHUNKS, unroll=8)
        def _(t):
            off = t * LANES
            v = vec_v[pl.ds(off, LANES)]
            ix = idx_v[pl.ds(off, LANES)]
            plsc.store_scatter(out_v, [ix >> 8, ix & 0xFF], v)

        for c in range(SIZE // LANES):
            ivec = lane + LANES * c
            src = (ivec * (ivec + 3)) >> 1
            d = plsc.load_gather(vec_v, [src])
            d = jnp.abs(d) + MIN_DIAG
            plsc.store_scatter(out_v, [ivec, ivec], d)

        pltpu.sync_copy(out_v, out_hbm.at[b])
        return 0

    lax.fori_loop(0, BPW, per_batch, 0)


def kernel(L_vec):
    return _tril_assemble(L_vec, jnp.asarray(_IJ_NP))


# speedup vs baseline: 5.1758x; 1.3576x over previous
import functools

import jax
import jax.numpy as jnp
import numpy as np
from jax import lax
from jax.experimental import pallas as pl
from jax.experimental.pallas import tpu as pltpu
from jax.experimental.pallas import tpu_sc as plsc

BATCH = 1024
SIZE = 128
NVEC = SIZE * (SIZE + 1) // 2
LANES = 16
NCHUNKS = NVEC // LANES
NW = 32
BPW = BATCH // NW
MIN_DIAG = 1e-8

_TI, _TJ = np.tril_indices(SIZE)
_IJ_NP = ((_TI << 8) | _TJ).astype(np.int32)

_MESH = plsc.VectorSubcoreMesh(core_axis_name="c", subcore_axis_name="s")


@functools.partial(
    pl.kernel,
    out_type=jax.ShapeDtypeStruct((BATCH, SIZE, SIZE), jnp.float32),
    mesh=_MESH,
    scratch_types=[
        pltpu.VMEM((NVEC,), jnp.int32),
        pltpu.VMEM((NVEC,), jnp.float32),
        pltpu.VMEM((NVEC,), jnp.float32),
        pltpu.VMEM((SIZE, SIZE), jnp.float32),
        pltpu.VMEM((SIZE, SIZE), jnp.float32),
        pltpu.SemaphoreType.DMA,
        pltpu.SemaphoreType.DMA,
        pltpu.SemaphoreType.DMA,
        pltpu.SemaphoreType.DMA,
    ],
    compiler_params=pltpu.CompilerParams(needs_layout_passes=False),
)
def _tril_assemble(vec_hbm, ij_hbm, out_hbm, idx_v, vec0, vec1, out0, out1,
                   insem0, insem1, outsem0, outsem1):
    wid = lax.axis_index("s") * 2 + lax.axis_index("c")
    base = wid * BPW
    lane = lax.iota(jnp.int32, LANES)
    zeros = jnp.zeros((LANES,), jnp.float32)
    vecs = (vec0, vec1)
    outs = (out0, out1)
    insems = (insem0, insem1)
    outsems = (outsem0, outsem1)

    pltpu.async_copy(vec_hbm.at[base], vec0, insem0)
    pltpu.sync_copy(ij_hbm, idx_v)

    for out_v in outs:
        @plsc.parallel_loop(0, SIZE, unroll=2)
        def _(r, out_v=out_v):
            for c in range(SIZE // LANES):
                out_v[r, pl.ds(c * LANES, LANES)] = zeros

    def pair_body(kk, _):
        for s in range(2):
            k = 2 * kk + s
            b = base + k
            vec_v = vecs[s]
            out_v = outs[s]

            pltpu.make_async_copy(vec_hbm.at[b], vec_v, insems[s]).wait()

            @pl.when(k < BPW - 1)
            def _():
                pltpu.async_copy(vec_hbm.at[b + 1], vecs[1 - s],
                                 insems[1 - s])

            @pl.when(kk > 0)
            def _():
                pltpu.make_async_copy(out_v, out_hbm.at[b - 2],
                                      outsems[s]).wait()

            @plsc.parallel_loop(0, NCHUNKS, unroll=8)
            def _(t):
                off = t * LANES
                v = vec_v[pl.ds(off, LANES)]
                ix = idx_v[pl.ds(off, LANES)]
                plsc.store_scatter(out_v, [ix >> 8, ix & 0xFF], v)

            for c in range(SIZE // LANES):
                ivec = lane + LANES * c
                src = (ivec * (ivec + 3)) >> 1
                d = plsc.load_gather(vec_v, [src])
                d = jnp.abs(d) + MIN_DIAG
                plsc.store_scatter(out_v, [ivec, ivec], d)

            pltpu.async_copy(out_v, out_hbm.at[b], outsems[s])
        return 0

    lax.fori_loop(0, BPW // 2, pair_body, 0)

    pltpu.make_async_copy(out0, out_hbm.at[base + BPW - 2], outsem0).wait()
    pltpu.make_async_copy(out1, out_hbm.at[base + BPW - 1], outsem1).wait()


def kernel(L_vec):
    return _tril_assemble(L_vec, jnp.asarray(_IJ_NP))
